# baseline (speedup 1.0000x reference)
import functools

import jax
import jax.numpy as jnp
from jax import lax
from jax.experimental import pallas as pl
from jax.experimental.pallas import tpu as pltpu
from jax.experimental.pallas import tpu_sc as plsc

_NC = 2
_NS = 16
_NW = _NC * _NS
_L = 16
_UNROLL = 8


@functools.lru_cache(maxsize=None)
def _build_kernel(V: int, D: int, BATCH: int, HIST: int):
    d_per_w = D // _NW
    mesh = plsc.VectorSubcoreMesh(core_axis_name="c", subcore_axis_name="s")

    @functools.partial(
        pl.kernel,
        out_type=jax.ShapeDtypeStruct((HIST, D, BATCH), jnp.float32),
        mesh=mesh,
        scratch_types=[
            pltpu.VMEM((V,), jnp.float32),
            pltpu.VMEM((4, 1, BATCH), jnp.int32),
            pltpu.VMEM((3, 1, 1, BATCH), jnp.float32),
            pltpu.SemaphoreType.DMA,
            pltpu.SemaphoreType.DMA,
            pltpu.SemaphoreType.DMA,
            pltpu.SemaphoreType.DMA,
            pltpu.SemaphoreType.DMA,
            pltpu.SemaphoreType.DMA,
            pltpu.SemaphoreType.DMA,
        ],
        compiler_params=pltpu.CompilerParams(needs_layout_passes=False),
    )
    def k(idx_t, table_t, out_t, row_v, idx_b, cell_b,
          isem0, isem1, isem2, isem3, osem0, osem1, osem2):
        wid = lax.axis_index("s") * _NC + lax.axis_index("c")
        d0 = wid * d_per_w
        isems = (isem0, isem1, isem2, isem3)
        osems = (osem0, osem1, osem2)

        def gather_cell(idx_v, cell_v):
            def per_vec(i, c2):
                offs = [(i * _UNROLL + u) * _L for u in range(_UNROLL)]
                vidx = [idx_v[pl.ds(o, _L)] for o in offs]
                vals = [plsc.load_gather(row_v, [vi]) for vi in vidx]
                for o, va in zip(offs, vals):
                    cell_v[pl.ds(o, _L)] = va
                return c2

            lax.fori_loop(0, BATCH // (_L * _UNROLL), per_vec, 0)

        def step(h, i, par, cb, d, first):
            pltpu.make_async_copy(
                idx_t.at[pl.ds(h, 1)], idx_b.at[par], isems[par]).wait()

            def drain():
                pltpu.make_async_copy(
                    cell_b.at[cb], out_t.at[pl.ds(h - 3, 1), pl.ds(d, 1)],
                    osems[cb]).wait()

            if first:
                pl.when(i > 0)(drain)
            else:
                drain()

            gather_cell(idx_b.at[par, 0], cell_b.at[cb, 0, 0])
            pltpu.async_copy(
                cell_b.at[cb], out_t.at[pl.ds(h, 1), pl.ds(d, 1)], osems[cb])

            @pl.when(h + 4 < HIST)
            def _():
                pltpu.async_copy(
                    idx_t.at[pl.ds(h + 4, 1)], idx_b.at[par], isems[par])

        def per_feature(df, carry):
            d = d0 + df
            pltpu.sync_copy(table_t.at[d], row_v)
            for par in range(4):
                pltpu.async_copy(
                    idx_t.at[pl.ds(par, 1)], idx_b.at[par], isems[par])

            def twelve(i, c):
                for u in range(12):
                    h = i * 12 + u
                    step(h, i, u % 4, u % 3, d, first=(u < 3))
                return c

            lax.fori_loop(0, HIST // 12, twelve, 0)
            for u in range(HIST % 12):
                h = HIST - (HIST % 12) + u
                step(h, HIST // 12, h % 4, h % 3, d, first=False)
            for h in range(HIST - 3, HIST):
                pltpu.make_async_copy(
                    cell_b.at[h % 3], out_t.at[pl.ds(h, 1), pl.ds(d, 1)],
                    osems[h % 3]).wait()
            return carry

        lax.fori_loop(0, d_per_w, per_feature, 0)

    return k


def kernel(inputs, embedding_weight):
    batch, hist = inputs.shape
    vocab, dim = embedding_weight.shape
    idx_t = inputs.T.astype(jnp.int32)
    table_t = embedding_weight.T
    out_t = _build_kernel(vocab, dim, batch, hist)(idx_t, table_t)
    return jnp.transpose(out_t, (2, 0, 1))


# speedup vs baseline: 1.0215x; 1.0215x over previous
import functools

import jax
import jax.numpy as jnp
from jax import lax
from jax.experimental import pallas as pl
from jax.experimental.pallas import tpu as pltpu
from jax.experimental.pallas import tpu_sc as plsc

_NC = 2
_NS = 16
_NW = _NC * _NS
_L = 16
_UNROLL = 8


@functools.lru_cache(maxsize=None)
def _build_kernel(V: int, D: int, BATCH: int, HIST: int):
    d_per_w = D // _NW
    mesh = plsc.VectorSubcoreMesh(core_axis_name="c", subcore_axis_name="s")

    @functools.partial(
        pl.kernel,
        out_type=jax.ShapeDtypeStruct((HIST, D, BATCH), jnp.float32),
        mesh=mesh,
        scratch_types=[
            pltpu.VMEM((V,), jnp.float32),
            pltpu.VMEM((4, 1, BATCH), jnp.int32),
            pltpu.VMEM((2, 1, 1, BATCH), jnp.float32),
            pltpu.SemaphoreType.DMA,
            pltpu.SemaphoreType.DMA,
            pltpu.SemaphoreType.DMA,
            pltpu.SemaphoreType.DMA,
            pltpu.SemaphoreType.DMA,
            pltpu.SemaphoreType.DMA,
        ],
        compiler_params=pltpu.CompilerParams(needs_layout_passes=False),
    )
    def k(idx_t, table_t, out_t, row_v, idx_b, cell_b,
          isem0, isem1, isem2, isem3, osem0, osem1):
        wid = lax.axis_index("s") * _NC + lax.axis_index("c")
        d0 = wid * d_per_w
        isems = (isem0, isem1, isem2, isem3)
        osems = (osem0, osem1)

        def gather_cell(idx_v, cell_v):
            def per_vec(i, c2):
                offs = [(i * _UNROLL + u) * _L for u in range(_UNROLL)]
                vidx = [idx_v[pl.ds(o, _L)] for o in offs]
                vals = [plsc.load_gather(row_v, [vi]) for vi in vidx]
                for o, va in zip(offs, vals):
                    cell_v[pl.ds(o, _L)] = va
                return c2

            lax.fori_loop(0, BATCH // (_L * _UNROLL), per_vec, 0)

        def step(h, i, par, d, first):
            cb = par % 2
            pltpu.make_async_copy(
                idx_t.at[pl.ds(h, 1)], idx_b.at[par], isems[par]).wait()

            def drain():
                pltpu.make_async_copy(
                    cell_b.at[cb], out_t.at[pl.ds(h - 2, 1), pl.ds(d, 1)],
                    osems[cb]).wait()

            if first:
                pl.when(i > 0)(drain)
            else:
                drain()

            gather_cell(idx_b.at[par, 0], cell_b.at[cb, 0, 0])
            pltpu.async_copy(
                cell_b.at[cb], out_t.at[pl.ds(h, 1), pl.ds(d, 1)], osems[cb])

            @pl.when(h + 4 < HIST)
            def _():
                pltpu.async_copy(
                    idx_t.at[pl.ds(h + 4, 1)], idx_b.at[par], isems[par])

        def per_feature(df, carry):
            d = d0 + df
            pltpu.sync_copy(table_t.at[d], row_v)
            for par in range(4):
                pltpu.async_copy(
                    idx_t.at[pl.ds(par, 1)], idx_b.at[par], isems[par])

            def quad(i, c):
                for par in range(4):
                    step(i * 4 + par, i, par, d, first=(par < 2))
                return c

            lax.fori_loop(0, HIST // 4, quad, 0)
            for par in range(HIST % 4):
                step(HIST - (HIST % 4) + par, HIST // 4, par, d, first=False)
            pltpu.make_async_copy(
                cell_b.at[0], out_t.at[pl.ds(HIST - 2, 1), pl.ds(d, 1)],
                osems[0]).wait()
            pltpu.make_async_copy(
                cell_b.at[1], out_t.at[pl.ds(HIST - 1, 1), pl.ds(d, 1)],
                osems[1]).wait()
            return carry

        lax.fori_loop(0, d_per_w, per_feature, 0)

    return k


def kernel(inputs, embedding_weight):
    batch, hist = inputs.shape
    vocab, dim = embedding_weight.shape
    idx_t = inputs.T.astype(jnp.int32)
    table_t = embedding_weight.T
    out_t = _build_kernel(vocab, dim, batch, hist)(idx_t, table_t)
    return jnp.transpose(out_t, (2, 0, 1))
